# baseline (speedup 1.0000x reference)
import functools

import jax
import jax.numpy as jnp
from jax import lax
from jax.experimental import pallas as pl
from jax.experimental.pallas import tpu as pltpu
from jax.experimental.pallas import tpu_sc as plsc

N_IN = 16384
N_OUT = 16384
WIDTH = 64
BATCH = 32
ZERO_ROW = WIDTH * N_IN

IB = 128


def _prep_body(fwt_ref, fmt_ref, xt_ref, m_ref, rm_ref, x3_ref, idx_ref):
    w2 = fwt_ref[...] * fmt_ref[...]
    x3_ref[pl.ds(0, WIDTH), :, :] = w2 * xt_ref[...][None, :, :]
    x3_ref[pl.ds(WIDTH, 1), :, :] = jnp.zeros((1, IB, BATCH), jnp.float32)
    ki = lax.broadcasted_iota(jnp.int32, (IB, WIDTH), 1) * N_IN
    idx_ref[...] = jnp.where(rm_ref[...] != 0, ki + m_ref[...], ZERO_ROW)


def _prep(fwt3, fmt3, xt, m, rm):
    return pl.pallas_call(
        _prep_body,
        grid=(N_IN // IB,),
        in_specs=[
            pl.BlockSpec((WIDTH, IB, 1), lambda i: (0, i, 0)),
            pl.BlockSpec((WIDTH, IB, 1), lambda i: (0, i, 0)),
            pl.BlockSpec((IB, BATCH), lambda i: (i, 0)),
            pl.BlockSpec((IB, WIDTH), lambda i: (i, 0)),
            pl.BlockSpec((IB, WIDTH), lambda i: (i, 0)),
        ],
        out_specs=[
            pl.BlockSpec((WIDTH + 1, IB, BATCH), lambda i: (0, i, 0)),
            pl.BlockSpec((IB, WIDTH), lambda i: (i, 0)),
        ],
        out_shape=[
            jax.ShapeDtypeStruct((WIDTH + 1, N_IN, BATCH), jnp.float32),
            jax.ShapeDtypeStruct((N_OUT, WIDTH), jnp.int32),
        ],
    )(fwt3, fmt3, xt, m, rm)


NC, NS = 2, 16
NW = NC * NS
JPW = N_OUT // NW
CJ = 16
CHUNKS = JPW // CJ
ROWS = CJ * WIDTH
NSTREAM = 8
SROWS = ROWS // NSTREAM
IDXROWS = N_OUT * WIDTH // SROWS
RPW = IDXROWS // NW


@functools.partial(
    pl.kernel,
    out_type=jax.ShapeDtypeStruct((N_OUT, BATCH), jnp.float32),
    mesh=plsc.VectorSubcoreMesh(core_axis_name="c", subcore_axis_name="s"),
    compiler_params=pltpu.CompilerParams(use_tc_tiling_on_sc=False),
    scratch_types=[
        pltpu.VMEM((2, NSTREAM, SROWS), jnp.int32),
        pltpu.VMEM((2, ROWS, BATCH), jnp.float32),
        pltpu.VMEM((CJ, BATCH), jnp.float32),
        pltpu.SemaphoreType.DMA,
        pltpu.SemaphoreType.DMA,
    ],
)
def _pool(x2_hbm, idx_hbm, out_hbm, idx_v, g_v, out_v, sem0, sem1):
    wid = lax.axis_index("s") * NC + lax.axis_index("c")
    row0 = wid * RPW
    j0 = wid * JPW
    sems = (sem0, sem1)

    def fire(c, b):
        pltpu.sync_copy(idx_hbm.at[pl.ds(row0 + c * NSTREAM, NSTREAM)],
                        idx_v.at[b])
        for s in range(NSTREAM):
            pltpu.async_copy(x2_hbm.at[idx_v.at[b, s]],
                             g_v.at[b, pl.ds(s * SROWS, SROWS)], sems[b])

    def drain(b):
        for s in range(NSTREAM):
            pltpu.make_async_copy(x2_hbm.at[pl.ds(0, SROWS)],
                                  g_v.at[b, pl.ds(s * SROWS, SROWS)],
                                  sems[b]).wait()

    def compute(c, b):
        def jj_body(jj, carry):
            r0 = jj * WIDTH
            acc0 = jnp.zeros((16,), jnp.float32)
            acc1 = jnp.zeros((16,), jnp.float32)
            for k in range(WIDTH):
                acc0 = acc0 + g_v[b, r0 + k, pl.ds(0, 16)]
                acc1 = acc1 + g_v[b, r0 + k, pl.ds(16, 16)]
            out_v[jj, pl.ds(0, 16)] = acc0
            out_v[jj, pl.ds(16, 16)] = acc1
            return carry
        lax.fori_loop(0, CJ, jj_body, 0)
        pltpu.sync_copy(out_v, out_hbm.at[pl.ds(j0 + c * CJ, CJ)])

    fire(0, 0)

    def outer(g2, carry):
        for b in range(2):
            c = g2 * 2 + b

            @pl.when(c < CHUNKS - 1)
            def _fire_next():
                fire(c + 1, 1 - b)

            drain(b)
            compute(c, b)
        return carry

    lax.fori_loop(0, CHUNKS // 2, outer, 0)


def kernel(x, forward_weights, forward_mask, reverse_mask, output_mapping):
    fwt3 = forward_weights.T[:, :, None]
    fmt3 = forward_mask.T[:, :, None]
    xt = x.T
    x3, idx = _prep(fwt3, fmt3, xt, output_mapping, reverse_mask)
    x2 = x3.reshape((WIDTH + 1) * N_IN, BATCH)
    idxf = idx.reshape(IDXROWS, SROWS)
    out = _pool(x2, idxf)
    return out.T


# speedup vs baseline: 8.9643x; 8.9643x over previous
import functools

import jax
import jax.numpy as jnp
from jax import lax
from jax.experimental import pallas as pl
from jax.experimental.pallas import tpu as pltpu
from jax.experimental.pallas import tpu_sc as plsc

N_IN = 16384
N_OUT = 16384
WIDTH = 64
BATCH = 32

NC, NS = 2, 16
NW = NC * NS
JPW = N_OUT // NW
CJ = 8
CHUNKS = JPW // CJ
ROWS = CJ * WIDTH
NSTREAM = 4
SROWS = ROWS // NSTREAM
IDXROWS = N_OUT * WIDTH // SROWS
RPW = IDXROWS // NW
XPW = N_IN // NS
KPT = WIDTH // NS
PIECES = 8
PC = N_IN // PIECES
PWORDS = PC // 2


@functools.partial(
    pl.kernel,
    out_type=jax.ShapeDtypeStruct((N_OUT, BATCH), jnp.float32),
    mesh=plsc.VectorSubcoreMesh(core_axis_name="c", subcore_axis_name="s"),
    compiler_params=pltpu.CompilerParams(use_tc_tiling_on_sc=False,
                                         needs_layout_passes=False),
    scratch_types=[
        pltpu.VMEM_SHARED((N_IN, BATCH), jnp.float32),
        pltpu.VMEM_SHARED((WIDTH, N_IN // 2), jnp.int32),
        pltpu.VMEM((N_IN,), jnp.float32),
        pltpu.VMEM((PC,), jnp.float32),
        pltpu.VMEM((PC,), jnp.float32),
        pltpu.VMEM((PC,), jnp.int32),
        pltpu.VMEM((PWORDS,), jnp.int32),
        pltpu.VMEM((2, NSTREAM, SROWS), jnp.int32),
        pltpu.VMEM((2, ROWS, BATCH), jnp.float32),
        pltpu.VMEM((2, WIDTH, CJ), jnp.int32),
        pltpu.VMEM((CJ, BATCH), jnp.float32),
        pltpu.SemaphoreType.DMA,
        pltpu.SemaphoreType.DMA,
    ],
)
def _densparse(xt_hbm, fwt_hbm, fmt_hbm, rmt_hbm, mt_hbm, idx_hbm, out_hbm,
               xt_sh, ct_sh, w2_v, fm_v, rm_v, mi_v, cw_v, idx_v, g_v,
               cstr_v, out_v, sem0, sem1):
    tid = lax.axis_index("s")
    wid = tid * NC + lax.axis_index("c")
    row0 = wid * RPW
    j0 = wid * JPW
    sems = (sem0, sem1)

    pltpu.sync_copy(xt_hbm.at[pl.ds(tid * XPW, XPW)],
                    xt_sh.at[pl.ds(tid * XPW, XPW)])

    for kk in range(KPT):
        k = tid * KPT + kk
        pltpu.sync_copy(fwt_hbm.at[k], w2_v)
        for h in range(PIECES):
            pltpu.sync_copy(fmt_hbm.at[k, pl.ds(h * PC, PC)], fm_v)

            def mul_body(i, carry):
                a = pl.ds(h * PC + i * 16, 16)
                w2_v[a] = w2_v[a] * fm_v[pl.ds(i * 16, 16)]
                return carry
            lax.fori_loop(0, PC // 16, mul_body, 0)
        for h in range(PIECES):
            pltpu.sync_copy(mt_hbm.at[k, pl.ds(h * PC, PC)], mi_v)
            pltpu.sync_copy(rmt_hbm.at[k, pl.ds(h * PC, PC)], rm_v)

            def gat_body(q, carry):
                a0 = pl.ds(q * 32, 16)
                a1 = pl.ds(q * 32 + 16, 16)
                v0 = plsc.load_gather(w2_v, [mi_v[a0]]) * rm_v[a0]
                v1 = plsc.load_gather(w2_v, [mi_v[a1]]) * rm_v[a1]
                packed = plsc.bitcast(
                    plsc.pack(v0, v1, format=plsc.PackFormat.INTERLEAVED),
                    jnp.int32)
                cw_v[pl.ds(q * 16, 16)] = packed
                return carry
            lax.fori_loop(0, PC // 32, gat_body, 0)
            pltpu.sync_copy(cw_v, ct_sh.at[k, pl.ds(h * PWORDS, PWORDS)])

    plsc.subcore_barrier()

    def fire(c, b):
        pltpu.sync_copy(idx_hbm.at[pl.ds(row0 + c * NSTREAM, NSTREAM)],
                        idx_v.at[b])
        for s in range(NSTREAM):
            pltpu.async_copy(xt_sh.at[idx_v.at[b, s]],
                             g_v.at[b, pl.ds(s * SROWS, SROWS)], sems[b])
        ws = wid * 256 + lax.div(c, 4) * 16 + lax.rem(c, 2) * 8
        pltpu.async_copy(ct_sh.at[:, pl.ds(ws, CJ)],
                         cstr_v.at[b], sems[b])

    def drain(b):
        for s in range(NSTREAM):
            pltpu.make_async_copy(xt_hbm.at[pl.ds(0, SROWS)],
                                  g_v.at[b, pl.ds(s * SROWS, SROWS)],
                                  sems[b]).wait()
        pltpu.make_async_copy(idx_hbm.at[pl.ds(0, WIDTH), pl.ds(0, CJ)],
                              cstr_v.at[b], sems[b]).wait()

    lane = lax.broadcasted_iota(jnp.int32, (16,), 0)
    gd = lax.GatherDimensionNumbers(offset_dims=(), collapsed_slice_dims=(0,),
                                    start_index_map=(0,))
    himask = jnp.full((16,), -65536, jnp.int32)

    def take16(v, idx):
        return lax.gather(v, idx[:, None], gd, (1,),
                          mode=lax.GatherScatterMode.PROMISE_IN_BOUNDS)

    def compute(c, b):
        shift = (1 - lax.rem(lax.div(c, 2), 2)) * 16

        def jj_body(jj, carry):
            col = jnp.full((16,), 0, jnp.int32) + jj
            cf = []
            for g in range(WIDTH // 16):
                w = plsc.load_gather(cstr_v.at[b], [lane + 16 * g, col])
                bits = jnp.left_shift(w, shift) & himask
                cf.append(lax.bitcast_convert_type(bits, jnp.float32))
            r0 = jj * WIDTH
            acc0 = jnp.zeros((16,), jnp.float32)
            acc1 = jnp.zeros((16,), jnp.float32)
            for k in range(WIDTH):
                cb = take16(cf[k // 16], jnp.full((16,), k % 16, jnp.int32))
                acc0 = acc0 + cb * g_v[b, r0 + k, pl.ds(0, 16)]
                acc1 = acc1 + cb * g_v[b, r0 + k, pl.ds(16, 16)]
            out_v[jj, pl.ds(0, 16)] = acc0
            out_v[jj, pl.ds(16, 16)] = acc1
            return carry
        lax.fori_loop(0, CJ, jj_body, 0)
        pltpu.sync_copy(out_v, out_hbm.at[pl.ds(j0 + c * CJ, CJ)])

    fire(0, 0)

    def outer(g2, carry):
        for b in range(2):
            c = g2 * 2 + b

            @pl.when(c < CHUNKS - 1)
            def _fire_next():
                fire(c + 1, 1 - b)

            drain(b)
            compute(c, b)
        return carry

    lax.fori_loop(0, CHUNKS // 2, outer, 0)


def kernel(x, forward_weights, forward_mask, reverse_mask, output_mapping):
    xt = x.T
    fwt = forward_weights.T
    fmt = forward_mask.T
    rmt = reverse_mask.T
    mt = output_mapping.T
    idxf = output_mapping.reshape(IDXROWS, SROWS)
    out = _densparse(xt, fwt, fmt, rmt, mt, idxf)
    return out.T
